# baseline (speedup 1.0000x reference)
import functools

import jax
import jax.numpy as jnp
from jax import lax
from jax.experimental import pallas as pl
from jax.experimental.pallas import tpu as pltpu
from jax.experimental.pallas import tpu_sc as plsc

N = 10000
E = 320000
D = 128

NP = 10240
NW = 32
EK = 128
ECH = 80
K = 128
EPAD = NW * ECH * EK
PAD_NODE = 10016
TPB = NP // 16
NSTAGE = TPB // K

_f32 = jnp.float32
_i32 = jnp.int32


def _make_deg():
    mesh = plsc.VectorSubcoreMesh(core_axis_name="c", subcore_axis_name="s")

    @functools.partial(
        pl.kernel,
        mesh=mesh,
        out_type=(
            jax.ShapeDtypeStruct((NP,), _f32),
            jax.ShapeDtypeStruct((NP,), _f32),
        ),
        scratch_types=[
            pltpu.VMEM((ECH, EK), _i32),
            pltpu.VMEM((EK,), _f32),
            pltpu.VMEM((TPB,), _f32),
            pltpu.VMEM_SHARED((NP,), _f32),
            pltpu.SemaphoreType.DMA,
        ],
    )
    def deg_kernel(dstp, deg0, deg1, didx, ones, stag, degsh, sem):
        c = lax.axis_index("c")
        s = lax.axis_index("s")
        wid = s * 2 + c
        base = s * TPB

        zf = jnp.zeros((16,), _f32)
        onef = jnp.ones((16,), _f32)

        def zrow(i, carry):
            stag[pl.ds(i * 16, 16)] = zf
            return carry

        lax.fori_loop(0, TPB // 16, zrow, 0)

        def orow(i, carry):
            ones[pl.ds(i * 16, 16)] = onef
            return carry

        lax.fori_loop(0, EK // 16, orow, 0)

        pltpu.sync_copy(stag.at[:], degsh.at[pl.ds(base, TPB)])
        pltpu.sync_copy(dstp.at[wid], didx)
        plsc.subcore_barrier()

        def body(j, carry):
            pltpu.async_copy(ones, degsh.at[didx.at[j]], sem, add=True)
            return carry

        lax.fori_loop(0, ECH, body, 0)

        def drain(j, carry):
            pltpu.make_async_copy(ones, degsh.at[didx.at[0]], sem).wait()
            return carry

        lax.fori_loop(0, ECH, drain, 0)
        plsc.subcore_barrier()

        @pl.when(c == 0)
        def _():
            pltpu.sync_copy(degsh.at[pl.ds(base, TPB)], deg0.at[pl.ds(base, TPB)])

        @pl.when(c == 1)
        def _():
            pltpu.sync_copy(degsh.at[pl.ds(base, TPB)], deg1.at[pl.ds(base, TPB)])

    return deg_kernel


def _make_agg():
    mesh = plsc.VectorSubcoreMesh(core_axis_name="c", subcore_axis_name="s")

    @functools.partial(
        pl.kernel,
        mesh=mesh,
        out_type=(
            jax.ShapeDtypeStruct((NP, D), _f32),
            jax.ShapeDtypeStruct((NP, D), _f32),
        ),
        scratch_types=[
            pltpu.VMEM((ECH // 5, EK), _i32),
            pltpu.VMEM((ECH // 5, EK), _i32),
            pltpu.VMEM((ECH // 5, EK), _i32),
            pltpu.VMEM((ECH // 5, EK), _i32),
            pltpu.VMEM((EK, D), _f32),
            pltpu.VMEM((EK, D), _f32),
            pltpu.VMEM_SHARED((NP, D), _f32),
            pltpu.SemaphoreType.DMA,
            pltpu.SemaphoreType.DMA,
            pltpu.SemaphoreType.DMA,
        ],
    )
    def agg_kernel(g, srcp, dstp, out0, out1, sa_s, sa_d, sb_s, sb_d,
                   r0_, r1_, acc, g0_, g1_, slsem):
        c = lax.axis_index("c")
        s = lax.axis_index("s")
        wid = s * 2 + c
        base = s * TPB
        bufs = ((r0_, g0_), (r1_, g1_))
        slabs = ((sa_s, sa_d), (sb_s, sb_d))
        PH = ECH // 5

        @pl.when(c == 0)
        def _():
            pltpu.sync_copy(g.at[pl.ds(base, TPB)], acc.at[pl.ds(base, TPB)])

        @pl.when(c == 1)
        def _():
            zf = jnp.zeros((16,), _f32)

            def zrow(i, carry):
                for jj in range(D // 16):
                    r0_[i, pl.ds(jj * 16, 16)] = zf
                return carry

            lax.fori_loop(0, EK, zrow, 0)
            for off in range(0, TPB, EK):
                sz = min(EK, TPB - off)
                pltpu.sync_copy(r0_.at[pl.ds(0, sz)],
                                acc.at[pl.ds(base + off, sz)])

        pltpu.sync_copy(srcp.at[wid, pl.ds(0, PH)], sa_s)
        pltpu.sync_copy(dstp.at[wid, pl.ds(0, PH)], sa_d)
        plsc.subcore_barrier()

        for p in range(5):
            sidx, didx = slabs[p % 2]
            if p < 4:
                nsi, ndi = slabs[(p + 1) % 2]
                pltpu.async_copy(srcp.at[wid, pl.ds((p + 1) * PH, PH)],
                                 nsi, slsem)
                pltpu.async_copy(dstp.at[wid, pl.ds((p + 1) * PH, PH)],
                                 ndi, slsem)
            pltpu.async_copy(g.at[sidx.at[0]], r0_, g0_)
            pltpu.async_copy(g.at[sidx.at[1]], r1_, g1_)

            def outer(jj, carry):
                for b, (rows, gsem) in enumerate(bufs):
                    j = jj * 2 + b
                    pltpu.make_async_copy(g.at[sidx.at[j]], rows, gsem).wait()
                    pltpu.sync_copy(rows.at[:], acc.at[didx.at[j]], add=True)

                    @pl.when(j + 2 < PH)
                    def _():
                        pltpu.async_copy(g.at[sidx.at[j + 2]], rows, gsem)

                return carry

            lax.fori_loop(0, PH // 2, outer, 0)
            if p < 4:
                nsi, ndi = slabs[(p + 1) % 2]
                off = (p + 1) * PH
                pltpu.make_async_copy(srcp.at[wid, pl.ds(off, PH)],
                                      nsi, slsem).wait()
                pltpu.make_async_copy(dstp.at[wid, pl.ds(off, PH)],
                                      ndi, slsem).wait()

        plsc.subcore_barrier()

        @pl.when(c == 0)
        def _():
            pltpu.sync_copy(acc.at[pl.ds(base, TPB)], out0.at[pl.ds(base, TPB)])

        @pl.when(c == 1)
        def _():
            pltpu.sync_copy(acc.at[pl.ds(base, TPB)], out1.at[pl.ds(base, TPB)])

    return agg_kernel


_deg_kernel = _make_deg()
_agg_kernel = _make_agg()


def _sb_from_deg(d0blk, d1blk, m):
    srows = 1.0 / jnp.sqrt(1.0 + d0blk + d1blk)
    lane = lax.broadcasted_iota(_i32, (K, K), 1)
    sub = lax.broadcasted_iota(_i32, (K, K), 0)
    blocks = []
    for r in range(m):
        srow = srows[r : r + 1, :]
        sd = jnp.where(lane == sub, jnp.broadcast_to(srow, (K, K)), 0.0)
        blocks.append(
            jnp.broadcast_to(jnp.sum(sd, axis=1, keepdims=True), (K, K))
        )
    return jnp.concatenate(blocks, axis=0)


_MB = 8


def _g1_body(x, w, d0, d1, og):
    sbblk = _sb_from_deg(d0[...], d1[...], _MB)
    og[...] = sbblk * jnp.dot(x[...], w[...], preferred_element_type=_f32)


def _g1(xp, w, d0, d1):
    return pl.pallas_call(
        _g1_body,
        grid=(NP // (_MB * K),),
        in_specs=[
            pl.BlockSpec((_MB * K, D), lambda b: (b, 0)),
            pl.BlockSpec((D, D), lambda b: (0, 0)),
            pl.BlockSpec((_MB, K), lambda b: (b, 0)),
            pl.BlockSpec((_MB, K), lambda b: (b, 0)),
        ],
        out_specs=pl.BlockSpec((_MB * K, D), lambda b: (b, 0)),
        out_shape=jax.ShapeDtypeStruct((NP, D), _f32),
    )(xp, w, d0.reshape(NP // K, K), d1.reshape(NP // K, K))


def _g23_body(a0, a1, d0, d1, w, o):
    sbblk = _sb_from_deg(d0[...], d1[...], _MB)
    t = jnp.maximum(sbblk * (a0[...] + a1[...]), 0.0)
    o[...] = sbblk * jnp.dot(t, w[...], preferred_element_type=_f32)


def _g23(a0, a1, d0, d1, w):
    return pl.pallas_call(
        _g23_body,
        grid=(NP // (_MB * K),),
        in_specs=[
            pl.BlockSpec((_MB * K, D), lambda b: (b, 0)),
            pl.BlockSpec((_MB * K, D), lambda b: (b, 0)),
            pl.BlockSpec((_MB, K), lambda b: (b, 0)),
            pl.BlockSpec((_MB, K), lambda b: (b, 0)),
            pl.BlockSpec((D, D), lambda b: (0, 0)),
        ],
        out_specs=pl.BlockSpec((_MB * K, D), lambda b: (b, 0)),
        out_shape=jax.ShapeDtypeStruct((NP, D), _f32),
    )(a0, a1, d0.reshape(NP // K, K), d1.reshape(NP // K, K), w)


def _final_body(a0, a1, d0, d1, o):
    sbblk = _sb_from_deg(d0[...], d1[...], _MB)
    o[...] = sbblk * (a0[...] + a1[...])


def _final(a0, a1, d0, d1):
    return pl.pallas_call(
        _final_body,
        grid=(NP // (_MB * K),),
        in_specs=[
            pl.BlockSpec((_MB * K, D), lambda b: (b, 0)),
            pl.BlockSpec((_MB * K, D), lambda b: (b, 0)),
            pl.BlockSpec((_MB, K), lambda b: (b, 0)),
            pl.BlockSpec((_MB, K), lambda b: (b, 0)),
        ],
        out_specs=pl.BlockSpec((_MB * K, D), lambda b: (b, 0)),
        out_shape=jax.ShapeDtypeStruct((NP, D), _f32),
    )(a0, a1, d0.reshape(NP // K, K), d1.reshape(NP // K, K))


def kernel(x, edge_index, W1, W2, W3):
    src = edge_index[0].astype(_i32)
    dst = edge_index[1].astype(_i32)
    pad = PAD_NODE + (jnp.arange(EPAD - E, dtype=_i32) % (NP - PAD_NODE))
    srcp = jnp.concatenate([src, pad]).reshape(NW, ECH, EK)
    dstp = jnp.concatenate([dst, pad]).reshape(NW, ECH, EK)
    xp = jnp.pad(x, ((0, NP - N), (0, 0)))

    d0, d1 = _deg_kernel(dstp)
    g = _g1(xp, W1, d0, d1)
    a0, a1 = _agg_kernel(g, srcp, dstp)
    g = _g23(a0, a1, d0, d1, W2)
    a0, a1 = _agg_kernel(g, srcp, dstp)
    g = _g23(a0, a1, d0, d1, W3)
    a0, a1 = _agg_kernel(g, srcp, dstp)
    return _final(a0, a1, d0, d1)[:N]


# speedup vs baseline: 1.0308x; 1.0308x over previous
import functools

import jax
import jax.numpy as jnp
from jax import lax
from jax.experimental import pallas as pl
from jax.experimental.pallas import tpu as pltpu
from jax.experimental.pallas import tpu_sc as plsc

N = 10000
E = 320000
D = 128

NP = 10240
NW = 32
EK = 128
ECH = 80
K = 128
EPAD = NW * ECH * EK
PAD_NODE = 10016
TPB = NP // 16
NSTAGE = TPB // K

_f32 = jnp.float32
_i32 = jnp.int32


def _make_deg():
    mesh = plsc.VectorSubcoreMesh(core_axis_name="c", subcore_axis_name="s")

    @functools.partial(
        pl.kernel,
        mesh=mesh,
        out_type=(
            jax.ShapeDtypeStruct((NP,), _f32),
            jax.ShapeDtypeStruct((NP,), _f32),
        ),
        scratch_types=[
            pltpu.VMEM((ECH, EK), _i32),
            pltpu.VMEM((EK,), _f32),
            pltpu.VMEM((TPB,), _f32),
            pltpu.VMEM_SHARED((NP,), _f32),
            pltpu.SemaphoreType.DMA,
        ],
    )
    def deg_kernel(dstp, deg0, deg1, didx, ones, stag, degsh, sem):
        c = lax.axis_index("c")
        s = lax.axis_index("s")
        wid = s * 2 + c
        base = s * TPB

        zf = jnp.zeros((16,), _f32)
        onef = jnp.ones((16,), _f32)

        def zrow(i, carry):
            stag[pl.ds(i * 16, 16)] = zf
            return carry

        lax.fori_loop(0, TPB // 16, zrow, 0)

        def orow(i, carry):
            ones[pl.ds(i * 16, 16)] = onef
            return carry

        lax.fori_loop(0, EK // 16, orow, 0)

        pltpu.sync_copy(stag.at[:], degsh.at[pl.ds(base, TPB)])
        pltpu.sync_copy(dstp.at[wid], didx)
        plsc.subcore_barrier()

        def body(j, carry):
            pltpu.async_copy(ones, degsh.at[didx.at[j]], sem, add=True)
            return carry

        lax.fori_loop(0, ECH, body, 0)

        def drain(j, carry):
            pltpu.make_async_copy(ones, degsh.at[didx.at[0]], sem).wait()
            return carry

        lax.fori_loop(0, ECH, drain, 0)
        plsc.subcore_barrier()

        @pl.when(c == 0)
        def _():
            pltpu.sync_copy(degsh.at[pl.ds(base, TPB)], deg0.at[pl.ds(base, TPB)])

        @pl.when(c == 1)
        def _():
            pltpu.sync_copy(degsh.at[pl.ds(base, TPB)], deg1.at[pl.ds(base, TPB)])

    return deg_kernel


def _make_agg():
    mesh = plsc.VectorSubcoreMesh(core_axis_name="c", subcore_axis_name="s")

    @functools.partial(
        pl.kernel,
        mesh=mesh,
        out_type=(
            jax.ShapeDtypeStruct((NP, D), _f32),
            jax.ShapeDtypeStruct((NP, D), _f32),
        ),
        scratch_types=[
            pltpu.VMEM((ECH // 2, EK), _i32),
            pltpu.VMEM((ECH // 2, EK), _i32),
            pltpu.VMEM((EK, D), _f32),
            pltpu.VMEM((EK, D), _f32),
            pltpu.VMEM_SHARED((NP, D), _f32),
            pltpu.SemaphoreType.DMA,
            pltpu.SemaphoreType.DMA,
        ],
    )
    def agg_kernel(g, srcp, dstp, out0, out1, sidx, didx,
                   r0_, r1_, acc, g0_, g1_):
        c = lax.axis_index("c")
        s = lax.axis_index("s")
        wid = s * 2 + c
        base = s * TPB
        bufs = ((r0_, g0_), (r1_, g1_))
        PH = ECH // 2

        @pl.when(c == 0)
        def _():
            pltpu.sync_copy(g.at[pl.ds(base, TPB)], acc.at[pl.ds(base, TPB)])

        @pl.when(c == 1)
        def _():
            zf = jnp.zeros((16,), _f32)

            def zrow(i, carry):
                for jj in range(D // 16):
                    r0_[i, pl.ds(jj * 16, 16)] = zf
                return carry

            lax.fori_loop(0, EK, zrow, 0)
            for off in range(0, TPB, EK):
                sz = min(EK, TPB - off)
                pltpu.sync_copy(r0_.at[pl.ds(0, sz)],
                                acc.at[pl.ds(base + off, sz)])

        pltpu.sync_copy(srcp.at[wid, pl.ds(0, PH)], sidx)
        pltpu.sync_copy(dstp.at[wid, pl.ds(0, PH)], didx)
        plsc.subcore_barrier()

        for p in range(2):
            pltpu.async_copy(g.at[sidx.at[0]], r0_, g0_)
            pltpu.async_copy(g.at[sidx.at[1]], r1_, g1_)

            def outer(jj, carry):
                for b, (rows, gsem) in enumerate(bufs):
                    j = jj * 2 + b
                    pltpu.make_async_copy(g.at[sidx.at[j]], rows, gsem).wait()
                    pltpu.sync_copy(rows.at[:], acc.at[didx.at[j]], add=True)

                    @pl.when(j + 2 < PH)
                    def _():
                        pltpu.async_copy(g.at[sidx.at[j + 2]], rows, gsem)

                return carry

            lax.fori_loop(0, PH // 2, outer, 0)
            if p == 0:
                pltpu.sync_copy(srcp.at[wid, pl.ds(PH, PH)], sidx)
                pltpu.sync_copy(dstp.at[wid, pl.ds(PH, PH)], didx)

        plsc.subcore_barrier()

        @pl.when(c == 0)
        def _():
            pltpu.sync_copy(acc.at[pl.ds(base, TPB)], out0.at[pl.ds(base, TPB)])

        @pl.when(c == 1)
        def _():
            pltpu.sync_copy(acc.at[pl.ds(base, TPB)], out1.at[pl.ds(base, TPB)])

    return agg_kernel


_deg_kernel = _make_deg()
_agg_kernel = _make_agg()


def _sb_from_deg(d0blk, d1blk, m):
    srows = 1.0 / jnp.sqrt(1.0 + d0blk + d1blk)
    lane = lax.broadcasted_iota(_i32, (K, K), 1)
    sub = lax.broadcasted_iota(_i32, (K, K), 0)
    blocks = []
    for r in range(m):
        srow = srows[r : r + 1, :]
        sd = jnp.where(lane == sub, jnp.broadcast_to(srow, (K, K)), 0.0)
        blocks.append(
            jnp.broadcast_to(jnp.sum(sd, axis=1, keepdims=True), (K, K))
        )
    return jnp.concatenate(blocks, axis=0)


_MB = 8


def _g1_body(x, w, d0, d1, og):
    sbblk = _sb_from_deg(d0[...], d1[...], _MB)
    og[...] = sbblk * jnp.dot(x[...], w[...], preferred_element_type=_f32)


def _g1(xp, w, d0, d1):
    return pl.pallas_call(
        _g1_body,
        grid=(NP // (_MB * K),),
        in_specs=[
            pl.BlockSpec((_MB * K, D), lambda b: (b, 0)),
            pl.BlockSpec((D, D), lambda b: (0, 0)),
            pl.BlockSpec((_MB, K), lambda b: (b, 0)),
            pl.BlockSpec((_MB, K), lambda b: (b, 0)),
        ],
        out_specs=pl.BlockSpec((_MB * K, D), lambda b: (b, 0)),
        out_shape=jax.ShapeDtypeStruct((NP, D), _f32),
    )(xp, w, d0.reshape(NP // K, K), d1.reshape(NP // K, K))


def _g23_body(a0, a1, d0, d1, w, o):
    sbblk = _sb_from_deg(d0[...], d1[...], _MB)
    t = jnp.maximum(sbblk * (a0[...] + a1[...]), 0.0)
    o[...] = sbblk * jnp.dot(t, w[...], preferred_element_type=_f32)


def _g23(a0, a1, d0, d1, w):
    return pl.pallas_call(
        _g23_body,
        grid=(NP // (_MB * K),),
        in_specs=[
            pl.BlockSpec((_MB * K, D), lambda b: (b, 0)),
            pl.BlockSpec((_MB * K, D), lambda b: (b, 0)),
            pl.BlockSpec((_MB, K), lambda b: (b, 0)),
            pl.BlockSpec((_MB, K), lambda b: (b, 0)),
            pl.BlockSpec((D, D), lambda b: (0, 0)),
        ],
        out_specs=pl.BlockSpec((_MB * K, D), lambda b: (b, 0)),
        out_shape=jax.ShapeDtypeStruct((NP, D), _f32),
    )(a0, a1, d0.reshape(NP // K, K), d1.reshape(NP // K, K), w)


def _final_body(a0, a1, d0, d1, o):
    sbblk = _sb_from_deg(d0[...], d1[...], _MB)
    o[...] = sbblk * (a0[...] + a1[...])


def _final(a0, a1, d0, d1):
    return pl.pallas_call(
        _final_body,
        grid=(NP // (_MB * K),),
        in_specs=[
            pl.BlockSpec((_MB * K, D), lambda b: (b, 0)),
            pl.BlockSpec((_MB * K, D), lambda b: (b, 0)),
            pl.BlockSpec((_MB, K), lambda b: (b, 0)),
            pl.BlockSpec((_MB, K), lambda b: (b, 0)),
        ],
        out_specs=pl.BlockSpec((_MB * K, D), lambda b: (b, 0)),
        out_shape=jax.ShapeDtypeStruct((NP, D), _f32),
    )(a0, a1, d0.reshape(NP // K, K), d1.reshape(NP // K, K))


def kernel(x, edge_index, W1, W2, W3):
    src = edge_index[0].astype(_i32)
    dst = edge_index[1].astype(_i32)
    pad = PAD_NODE + (jnp.arange(EPAD - E, dtype=_i32) % (NP - PAD_NODE))
    srcp = jnp.concatenate([src, pad]).reshape(NW, ECH, EK)
    dstp = jnp.concatenate([dst, pad]).reshape(NW, ECH, EK)
    xp = jnp.pad(x, ((0, NP - N), (0, 0)))

    d0, d1 = _deg_kernel(dstp)
    g = _g1(xp, W1, d0, d1)
    a0, a1 = _agg_kernel(g, srcp, dstp)
    g = _g23(a0, a1, d0, d1, W2)
    a0, a1 = _agg_kernel(g, srcp, dstp)
    g = _g23(a0, a1, d0, d1, W3)
    a0, a1 = _agg_kernel(g, srcp, dstp)
    return _final(a0, a1, d0, d1)[:N]


# speedup vs baseline: 1.0420x; 1.0110x over previous
import functools

import jax
import jax.numpy as jnp
from jax import lax
from jax.experimental import pallas as pl
from jax.experimental.pallas import tpu as pltpu
from jax.experimental.pallas import tpu_sc as plsc

N = 10000
E = 320000
D = 128

NP = 10240
NW = 32
EK = 128
ECH = 80
K = 128
EPAD = NW * ECH * EK
PAD_NODE = 10016
TPB = NP // 16
NSTAGE = TPB // K

_f32 = jnp.float32
_i32 = jnp.int32


def _make_deg():
    mesh = plsc.VectorSubcoreMesh(core_axis_name="c", subcore_axis_name="s")

    @functools.partial(
        pl.kernel,
        mesh=mesh,
        out_type=(
            jax.ShapeDtypeStruct((NP,), _f32),
            jax.ShapeDtypeStruct((NP,), _f32),
        ),
        scratch_types=[
            pltpu.VMEM((ECH, EK), _i32),
            pltpu.VMEM((EK,), _f32),
            pltpu.VMEM((TPB,), _f32),
            pltpu.VMEM_SHARED((NP,), _f32),
            pltpu.SemaphoreType.DMA,
        ],
    )
    def deg_kernel(dstp, deg0, deg1, didx, ones, stag, degsh, sem):
        c = lax.axis_index("c")
        s = lax.axis_index("s")
        wid = s * 2 + c
        base = s * TPB

        zf = jnp.zeros((16,), _f32)
        onef = jnp.ones((16,), _f32)

        def zrow(i, carry):
            stag[pl.ds(i * 16, 16)] = zf
            return carry

        lax.fori_loop(0, TPB // 16, zrow, 0)

        def orow(i, carry):
            ones[pl.ds(i * 16, 16)] = onef
            return carry

        lax.fori_loop(0, EK // 16, orow, 0)

        pltpu.sync_copy(stag.at[:], degsh.at[pl.ds(base, TPB)])
        pltpu.sync_copy(dstp.at[wid], didx)
        plsc.subcore_barrier()

        def body(j, carry):
            pltpu.async_copy(ones, degsh.at[didx.at[j]], sem, add=True)
            return carry

        lax.fori_loop(0, ECH, body, 0)

        def drain(j, carry):
            pltpu.make_async_copy(ones, degsh.at[didx.at[0]], sem).wait()
            return carry

        lax.fori_loop(0, ECH, drain, 0)
        plsc.subcore_barrier()

        @pl.when(c == 0)
        def _():
            pltpu.sync_copy(degsh.at[pl.ds(base, TPB)], deg0.at[pl.ds(base, TPB)])

        @pl.when(c == 1)
        def _():
            pltpu.sync_copy(degsh.at[pl.ds(base, TPB)], deg1.at[pl.ds(base, TPB)])

    return deg_kernel


def _make_agg():
    mesh = plsc.VectorSubcoreMesh(core_axis_name="c", subcore_axis_name="s")

    @functools.partial(
        pl.kernel,
        mesh=mesh,
        out_type=(
            jax.ShapeDtypeStruct((NP, D), _f32),
            jax.ShapeDtypeStruct((NP, D), _f32),
        ),
        scratch_types=[
            pltpu.VMEM((ECH // 2, EK), _i32),
            pltpu.VMEM((ECH // 2, EK), _i32),
            pltpu.VMEM((EK, D), _f32),
            pltpu.VMEM((EK, D), _f32),
            pltpu.VMEM_SHARED((NP, D), _f32),
            pltpu.SemaphoreType.DMA,
            pltpu.SemaphoreType.DMA,
        ],
    )
    def agg_kernel(g, srcp, dstp, out0, out1, sidx, didx,
                   r0_, r1_, acc, g0_, g1_):
        c = lax.axis_index("c")
        s = lax.axis_index("s")
        wid = s * 2 + c
        base = s * TPB
        bufs = ((r0_, g0_), (r1_, g1_))
        PH = ECH // 2

        @pl.when(c == 0)
        def _():
            pltpu.sync_copy(g.at[pl.ds(base, TPB)], acc.at[pl.ds(base, TPB)])

        @pl.when(c == 1)
        def _():
            zf = jnp.zeros((16,), _f32)

            def zrow(i, carry):
                for jj in range(D // 16):
                    r0_[i, pl.ds(jj * 16, 16)] = zf
                return carry

            lax.fori_loop(0, EK, zrow, 0)
            for off in range(0, TPB, EK):
                sz = min(EK, TPB - off)
                pltpu.sync_copy(r0_.at[pl.ds(0, sz)],
                                acc.at[pl.ds(base + off, sz)])

        pltpu.sync_copy(srcp.at[wid, pl.ds(0, PH)], sidx)
        pltpu.sync_copy(dstp.at[wid, pl.ds(0, PH)], didx)
        plsc.subcore_barrier()

        for p in range(2):
            pltpu.async_copy(g.at[sidx.at[0]], r0_, g0_)
            pltpu.async_copy(g.at[sidx.at[1]], r1_, g1_)

            def outer(jj, carry):
                for b, (rows, gsem) in enumerate(bufs):
                    j = jj * 2 + b
                    pltpu.make_async_copy(g.at[sidx.at[j]], rows, gsem).wait()
                    pltpu.sync_copy(rows.at[:], acc.at[didx.at[j]], add=True)

                    @pl.when(j + 2 < PH)
                    def _():
                        pltpu.async_copy(g.at[sidx.at[j + 2]], rows, gsem)

                return carry

            lax.fori_loop(0, PH // 2, outer, 0)
            if p == 0:
                pltpu.sync_copy(srcp.at[wid, pl.ds(PH, PH)], sidx)
                pltpu.sync_copy(dstp.at[wid, pl.ds(PH, PH)], didx)

        plsc.subcore_barrier()

        @pl.when(c == 0)
        def _():
            pltpu.sync_copy(acc.at[pl.ds(base, TPB)], out0.at[pl.ds(base, TPB)])

        @pl.when(c == 1)
        def _():
            pltpu.sync_copy(acc.at[pl.ds(base, TPB)], out1.at[pl.ds(base, TPB)])

    return agg_kernel


_deg_kernel = _make_deg()
_agg_kernel = _make_agg()


def _sb_from_deg(d0blk, d1blk, m):
    srows = 1.0 / jnp.sqrt(1.0 + d0blk + d1blk)
    lane = lax.broadcasted_iota(_i32, (K, K), 1)
    sub = lax.broadcasted_iota(_i32, (K, K), 0)
    blocks = []
    for r in range(m):
        srow = srows[r : r + 1, :]
        sd = jnp.where(lane == sub, jnp.broadcast_to(srow, (K, K)), 0.0)
        blocks.append(
            jnp.broadcast_to(jnp.sum(sd, axis=1, keepdims=True), (K, K))
        )
    return jnp.concatenate(blocks, axis=0)


_MB = 8


def _g1_body(x, w, d0, d1, og):
    sbblk = _sb_from_deg(d0[...], d1[...], _MB)
    og[...] = sbblk * jnp.dot(x[...], w[...], preferred_element_type=_f32)


def _g1(xp, w, d0, d1):
    return pl.pallas_call(
        _g1_body,
        grid=(NP // (_MB * K),),
        in_specs=[
            pl.BlockSpec((_MB * K, D), lambda b: (b, 0)),
            pl.BlockSpec((D, D), lambda b: (0, 0)),
            pl.BlockSpec((_MB, K), lambda b: (b, 0)),
            pl.BlockSpec((_MB, K), lambda b: (b, 0)),
        ],
        out_specs=pl.BlockSpec((_MB * K, D), lambda b: (b, 0)),
        out_shape=jax.ShapeDtypeStruct((NP, D), _f32),
    )(xp, w, d0.reshape(NP // K, K), d1.reshape(NP // K, K))


def _g23_body(a0, a1, d0, d1, w, o):
    sbblk = _sb_from_deg(d0[...], d1[...], _MB)
    t = jnp.maximum(sbblk * (a0[...] + a1[...]), 0.0)
    o[...] = sbblk * jnp.dot(t, w[...], preferred_element_type=_f32)


def _g23(a0, a1, d0, d1, w):
    return pl.pallas_call(
        _g23_body,
        grid=(NP // (_MB * K),),
        in_specs=[
            pl.BlockSpec((_MB * K, D), lambda b: (b, 0)),
            pl.BlockSpec((_MB * K, D), lambda b: (b, 0)),
            pl.BlockSpec((_MB, K), lambda b: (b, 0)),
            pl.BlockSpec((_MB, K), lambda b: (b, 0)),
            pl.BlockSpec((D, D), lambda b: (0, 0)),
        ],
        out_specs=pl.BlockSpec((_MB * K, D), lambda b: (b, 0)),
        out_shape=jax.ShapeDtypeStruct((NP, D), _f32),
    )(a0, a1, d0.reshape(NP // K, K), d1.reshape(NP // K, K), w)


def _final_body(a0, a1, d0, d1, o):
    sbblk = _sb_from_deg(d0[...], d1[...], _MB)
    o[...] = sbblk * (a0[...] + a1[...])


def _final(a0, a1, d0, d1):
    return pl.pallas_call(
        _final_body,
        grid=(NP // (_MB * K),),
        in_specs=[
            pl.BlockSpec((_MB * K, D), lambda b: (b, 0)),
            pl.BlockSpec((_MB * K, D), lambda b: (b, 0)),
            pl.BlockSpec((_MB, K), lambda b: (b, 0)),
            pl.BlockSpec((_MB, K), lambda b: (b, 0)),
        ],
        out_specs=pl.BlockSpec((_MB * K, D), lambda b: (b, 0)),
        out_shape=jax.ShapeDtypeStruct((N, D), _f32),
    )(a0, a1, d0.reshape(NP // K, K), d1.reshape(NP // K, K))


def kernel(x, edge_index, W1, W2, W3):
    src = edge_index[0].astype(_i32)
    dst = edge_index[1].astype(_i32)
    pad = PAD_NODE + (jnp.arange(EPAD - E, dtype=_i32) % (NP - PAD_NODE))
    srcp = jnp.concatenate([src, pad]).reshape(NW, ECH, EK)
    dstp = jnp.concatenate([dst, pad]).reshape(NW, ECH, EK)
    d0, d1 = _deg_kernel(dstp)
    g = _g1(x, W1, d0, d1)
    a0, a1 = _agg_kernel(g, srcp, dstp)
    g = _g23(a0, a1, d0, d1, W2)
    a0, a1 = _agg_kernel(g, srcp, dstp)
    g = _g23(a0, a1, d0, d1, W3)
    a0, a1 = _agg_kernel(g, srcp, dstp)
    return _final(a0, a1, d0, d1)


# speedup vs baseline: 1.0555x; 1.0129x over previous
import functools

import jax
import jax.numpy as jnp
from jax import lax
from jax.experimental import pallas as pl
from jax.experimental.pallas import tpu as pltpu
from jax.experimental.pallas import tpu_sc as plsc

N = 10000
E = 320000
D = 128

NP = 10240
NW = 32
EK = 128
ECH = 80
K = 128
EPAD = NW * ECH * EK
PAD_NODE = 10016
TPB = NP // 16
NSTAGE = TPB // K

_f32 = jnp.float32
_i32 = jnp.int32


def _make_deg():
    mesh = plsc.VectorSubcoreMesh(core_axis_name="c", subcore_axis_name="s")

    @functools.partial(
        pl.kernel,
        mesh=mesh,
        out_type=(
            jax.ShapeDtypeStruct((NP,), _f32),
            jax.ShapeDtypeStruct((NP,), _f32),
        ),
        scratch_types=[
            pltpu.VMEM((ECH, EK), _i32),
            pltpu.VMEM((EK,), _f32),
            pltpu.VMEM((TPB,), _f32),
            pltpu.VMEM_SHARED((NP,), _f32),
            pltpu.SemaphoreType.DMA,
        ],
    )
    def deg_kernel(dstp, deg0, deg1, didx, ones, stag, degsh, sem):
        c = lax.axis_index("c")
        s = lax.axis_index("s")
        wid = s * 2 + c
        base = s * TPB

        zf = jnp.zeros((16,), _f32)
        onef = jnp.ones((16,), _f32)

        def zrow(i, carry):
            stag[pl.ds(i * 16, 16)] = zf
            return carry

        lax.fori_loop(0, TPB // 16, zrow, 0)

        def orow(i, carry):
            ones[pl.ds(i * 16, 16)] = onef
            return carry

        lax.fori_loop(0, EK // 16, orow, 0)

        pltpu.sync_copy(stag.at[:], degsh.at[pl.ds(base, TPB)])
        pltpu.sync_copy(dstp.at[wid], didx)
        plsc.subcore_barrier()

        def body(j, carry):
            pltpu.async_copy(ones, degsh.at[didx.at[j]], sem, add=True)
            return carry

        lax.fori_loop(0, ECH, body, 0)

        def drain(j, carry):
            pltpu.make_async_copy(ones, degsh.at[didx.at[0]], sem).wait()
            return carry

        lax.fori_loop(0, ECH, drain, 0)
        plsc.subcore_barrier()

        @pl.when(c == 0)
        def _():
            pltpu.sync_copy(degsh.at[pl.ds(base, TPB)], deg0.at[pl.ds(base, TPB)])

        @pl.when(c == 1)
        def _():
            pltpu.sync_copy(degsh.at[pl.ds(base, TPB)], deg1.at[pl.ds(base, TPB)])

    return deg_kernel


def _make_agg():
    mesh = plsc.VectorSubcoreMesh(core_axis_name="c", subcore_axis_name="s")

    @functools.partial(
        pl.kernel,
        mesh=mesh,
        out_type=(
            jax.ShapeDtypeStruct((NP, D), _f32),
            jax.ShapeDtypeStruct((NP, D), _f32),
        ),
        scratch_types=[
            pltpu.VMEM((ECH // 2, EK), _i32),
            pltpu.VMEM((ECH // 2, EK), _i32),
            pltpu.VMEM((EK, D), _f32),
            pltpu.VMEM((EK, D), _f32),
            pltpu.VMEM_SHARED((NP, D), _f32),
            pltpu.SemaphoreType.DMA,
            pltpu.SemaphoreType.DMA,
        ],
    )
    def agg_kernel(g, srcp, dstp, out0, out1, sidx, didx,
                   r0_, r1_, acc, g0_, g1_):
        c = lax.axis_index("c")
        s = lax.axis_index("s")
        wid = s * 2 + c
        base = s * TPB
        bufs = ((r0_, g0_), (r1_, g1_))
        PH = ECH // 2

        @pl.when(c == 0)
        def _():
            pltpu.async_copy(g.at[pl.ds(base, TPB)],
                             acc.at[pl.ds(base, TPB)], g0_)

        @pl.when(c == 1)
        def _():
            zf = jnp.zeros((16,), _f32)

            def zrow(i, carry):
                for jj in range(D // 16):
                    r0_[i, pl.ds(jj * 16, 16)] = zf
                return carry

            lax.fori_loop(0, EK, zrow, 0)
            for off in range(0, TPB, EK):
                sz = min(EK, TPB - off)
                pltpu.async_copy(r0_.at[pl.ds(0, sz)],
                                 acc.at[pl.ds(base + off, sz)], g0_)

        pltpu.sync_copy(srcp.at[wid, pl.ds(0, PH)], sidx)
        pltpu.sync_copy(dstp.at[wid, pl.ds(0, PH)], didx)

        @pl.when(c == 0)
        def _():
            pltpu.make_async_copy(g.at[pl.ds(base, TPB)],
                                  acc.at[pl.ds(base, TPB)], g0_).wait()

        @pl.when(c == 1)
        def _():
            for off in range(0, TPB, EK):
                sz = min(EK, TPB - off)
                pltpu.make_async_copy(r0_.at[pl.ds(0, sz)],
                                      acc.at[pl.ds(base + off, sz)],
                                      g0_).wait()

        plsc.subcore_barrier()

        for p in range(2):
            pltpu.async_copy(g.at[sidx.at[0]], r0_, g0_)
            pltpu.async_copy(g.at[sidx.at[1]], r1_, g1_)

            def outer(jj, carry):
                for b, (rows, gsem) in enumerate(bufs):
                    j = jj * 2 + b
                    pltpu.make_async_copy(g.at[sidx.at[j]], rows, gsem).wait()
                    pltpu.sync_copy(rows.at[:], acc.at[didx.at[j]], add=True)

                    @pl.when(j + 2 < PH)
                    def _():
                        pltpu.async_copy(g.at[sidx.at[j + 2]], rows, gsem)

                return carry

            lax.fori_loop(0, PH // 2, outer, 0)
            if p == 0:
                pltpu.sync_copy(srcp.at[wid, pl.ds(PH, PH)], sidx)
                pltpu.sync_copy(dstp.at[wid, pl.ds(PH, PH)], didx)

        plsc.subcore_barrier()

        @pl.when(c == 0)
        def _():
            pltpu.sync_copy(acc.at[pl.ds(base, TPB)], out0.at[pl.ds(base, TPB)])

        @pl.when(c == 1)
        def _():
            pltpu.sync_copy(acc.at[pl.ds(base, TPB)], out1.at[pl.ds(base, TPB)])

    return agg_kernel


_deg_kernel = _make_deg()
_agg_kernel = _make_agg()


def _sb_from_deg(d0blk, d1blk, m):
    srows = 1.0 / jnp.sqrt(1.0 + d0blk + d1blk)
    lane = lax.broadcasted_iota(_i32, (K, K), 1)
    sub = lax.broadcasted_iota(_i32, (K, K), 0)
    blocks = []
    for r in range(m):
        srow = srows[r : r + 1, :]
        sd = jnp.where(lane == sub, jnp.broadcast_to(srow, (K, K)), 0.0)
        blocks.append(
            jnp.broadcast_to(jnp.sum(sd, axis=1, keepdims=True), (K, K))
        )
    return jnp.concatenate(blocks, axis=0)


_MB = 8


def _g1_body(x, w, d0, d1, og):
    sbblk = _sb_from_deg(d0[...], d1[...], _MB)
    og[...] = sbblk * jnp.dot(x[...], w[...], preferred_element_type=_f32)


def _g1(xp, w, d0, d1):
    return pl.pallas_call(
        _g1_body,
        grid=(NP // (_MB * K),),
        in_specs=[
            pl.BlockSpec((_MB * K, D), lambda b: (b, 0)),
            pl.BlockSpec((D, D), lambda b: (0, 0)),
            pl.BlockSpec((_MB, K), lambda b: (b, 0)),
            pl.BlockSpec((_MB, K), lambda b: (b, 0)),
        ],
        out_specs=pl.BlockSpec((_MB * K, D), lambda b: (b, 0)),
        out_shape=jax.ShapeDtypeStruct((NP, D), _f32),
    )(xp, w, d0.reshape(NP // K, K), d1.reshape(NP // K, K))


def _g23_body(a0, a1, d0, d1, w, o):
    sbblk = _sb_from_deg(d0[...], d1[...], _MB)
    t = jnp.maximum(sbblk * (a0[...] + a1[...]), 0.0)
    o[...] = sbblk * jnp.dot(t, w[...], preferred_element_type=_f32)


def _g23(a0, a1, d0, d1, w):
    return pl.pallas_call(
        _g23_body,
        grid=(NP // (_MB * K),),
        in_specs=[
            pl.BlockSpec((_MB * K, D), lambda b: (b, 0)),
            pl.BlockSpec((_MB * K, D), lambda b: (b, 0)),
            pl.BlockSpec((_MB, K), lambda b: (b, 0)),
            pl.BlockSpec((_MB, K), lambda b: (b, 0)),
            pl.BlockSpec((D, D), lambda b: (0, 0)),
        ],
        out_specs=pl.BlockSpec((_MB * K, D), lambda b: (b, 0)),
        out_shape=jax.ShapeDtypeStruct((NP, D), _f32),
    )(a0, a1, d0.reshape(NP // K, K), d1.reshape(NP // K, K), w)


def _final_body(a0, a1, d0, d1, o):
    sbblk = _sb_from_deg(d0[...], d1[...], _MB)
    o[...] = sbblk * (a0[...] + a1[...])


def _final(a0, a1, d0, d1):
    return pl.pallas_call(
        _final_body,
        grid=(NP // (_MB * K),),
        in_specs=[
            pl.BlockSpec((_MB * K, D), lambda b: (b, 0)),
            pl.BlockSpec((_MB * K, D), lambda b: (b, 0)),
            pl.BlockSpec((_MB, K), lambda b: (b, 0)),
            pl.BlockSpec((_MB, K), lambda b: (b, 0)),
        ],
        out_specs=pl.BlockSpec((_MB * K, D), lambda b: (b, 0)),
        out_shape=jax.ShapeDtypeStruct((N, D), _f32),
    )(a0, a1, d0.reshape(NP // K, K), d1.reshape(NP // K, K))


def kernel(x, edge_index, W1, W2, W3):
    src = edge_index[0].astype(_i32)
    dst = edge_index[1].astype(_i32)
    pad = PAD_NODE + (jnp.arange(EPAD - E, dtype=_i32) % (NP - PAD_NODE))
    srcp = jnp.concatenate([src, pad]).reshape(NW, ECH, EK)
    dstp = jnp.concatenate([dst, pad]).reshape(NW, ECH, EK)
    d0, d1 = _deg_kernel(dstp)
    g = _g1(x, W1, d0, d1)
    a0, a1 = _agg_kernel(g, srcp, dstp)
    g = _g23(a0, a1, d0, d1, W2)
    a0, a1 = _agg_kernel(g, srcp, dstp)
    g = _g23(a0, a1, d0, d1, W3)
    a0, a1 = _agg_kernel(g, srcp, dstp)
    return _final(a0, a1, d0, d1)
